# baseline (speedup 1.0000x reference)
import functools

import jax
import jax.numpy as jnp
from jax import lax
from jax.experimental import pallas as pl
from jax.experimental.pallas import tpu as pltpu
from jax.experimental.pallas import tpu_sc as plsc

B = 4096
F = 26
D = 128
VOCAB_PER_FIELD = 100000

NC = 2
NS = 16
L = 16
NW = NC * NS
BPW = B // NW
FLAT = BPW * F
C = 4
ROWS = C * F
NCHUNK = BPW // C
NBUF = 8

_mesh = plsc.VectorSubcoreMesh(core_axis_name="c", subcore_axis_name="s")


@functools.partial(
    pl.kernel,
    out_type=jax.ShapeDtypeStruct((B, D), jnp.float32),
    mesh=_mesh,
    scratch_types=[
        pltpu.VMEM((FLAT,), jnp.int32),
        pltpu.VMEM((NBUF, ROWS, D), jnp.float32),
        pltpu.VMEM((BPW, D), jnp.float32),
        pltpu.SemaphoreType.DMA,
        pltpu.SemaphoreType.DMA,
        pltpu.SemaphoreType.DMA,
        pltpu.SemaphoreType.DMA,
        pltpu.SemaphoreType.DMA,
        pltpu.SemaphoreType.DMA,
        pltpu.SemaphoreType.DMA,
        pltpu.SemaphoreType.DMA,
    ],
)
def _emb_bag(sf_hbm, table_hbm, out_hbm, idx_v, rows_v, out_v, *sems):
    wid = lax.axis_index("s") * NC + lax.axis_index("c")
    base = wid * FLAT

    pltpu.sync_copy(sf_hbm.at[pl.ds(base, FLAT)], idx_v)

    def fix(i, _):
        off = i * L
        pos = off + lax.iota(jnp.int32, L)
        f = lax.rem(pos, F)
        idx_v[pl.ds(off, L)] = idx_v[pl.ds(off, L)] + f * VOCAB_PER_FIELD
        return 0

    def fire(j, slot):
        pltpu.async_copy(
            table_hbm.at[idx_v.at[pl.ds(j * ROWS, ROWS)]],
            rows_v.at[slot],
            sems[slot],
        )

    def drain(j, slot):
        pltpu.make_async_copy(
            table_hbm.at[idx_v.at[pl.ds(j * ROWS, ROWS)]],
            rows_v.at[slot],
            sems[slot],
        ).wait()

    def accumulate(j, slot):
        G = 4

        def bag(b, _):
            r0 = b * F
            for d0 in range(0, D // L, G):
                ds_ = [pl.ds((d0 + g) * L, L) for g in range(G)]
                accs = [rows_v[slot, r0, dd] for dd in ds_]
                for f in range(1, F):
                    for g in range(G):
                        accs[g] = accs[g] + rows_v[slot, r0 + f, ds_[g]]
                for g in range(G):
                    out_v[j * C + b, ds_[g]] = accs[g] * (1.0 / F)
            return 0

        lax.fori_loop(0, C, bag, 0)

    VPC = -(-ROWS // L)
    lax.fori_loop(0, VPC, fix, 0)
    fire(0, 0)
    lax.fori_loop(VPC, FLAT // L, fix, 0)

    for s in range(1, NBUF - 1):
        fire(s, s)

    def ring(j4, _):
        j0 = NBUF * j4
        for s in range(NBUF):
            j = j0 + s

            @pl.when(j + NBUF - 1 < NCHUNK)
            def _():
                fire(j + NBUF - 1, (s + NBUF - 1) % NBUF)

            drain(j, s)
            accumulate(j, s)
        return 0

    lax.fori_loop(0, NCHUNK // NBUF, ring, 0)

    pltpu.sync_copy(out_v, out_hbm.at[pl.ds(wid * BPW, BPW)])


def kernel(sparse_features, table):
    sf_flat = sparse_features.astype(jnp.int32).reshape(-1)
    return _emb_bag(sf_flat, table)


# speedup vs baseline: 1.0967x; 1.0967x over previous
import functools

import jax
import jax.numpy as jnp
from jax import lax
from jax.experimental import pallas as pl
from jax.experimental.pallas import tpu as pltpu
from jax.experimental.pallas import tpu_sc as plsc

B = 4096
F = 26
D = 128
VOCAB_PER_FIELD = 100000

NC = 2
NS = 16
L = 16
NW = NC * NS
BPW = B // NW
FLAT = BPW * F
C = 4
ROWS = C * F
NCHUNK = BPW // C
NBUF = 4

_mesh = plsc.VectorSubcoreMesh(core_axis_name="c", subcore_axis_name="s")


@functools.partial(
    pl.kernel,
    out_type=jax.ShapeDtypeStruct((B, D), jnp.float32),
    mesh=_mesh,
    scratch_types=[
        pltpu.VMEM((FLAT,), jnp.int32),
        pltpu.VMEM((NBUF, ROWS, D), jnp.float32),
        pltpu.VMEM((BPW, D), jnp.float32),
        pltpu.SemaphoreType.DMA,
        pltpu.SemaphoreType.DMA,
        pltpu.SemaphoreType.DMA,
        pltpu.SemaphoreType.DMA,
        pltpu.SemaphoreType.DMA,
    ],
)
def _emb_bag(sf_hbm, table_hbm, out_hbm, idx_v, rows_v, out_v, *sems):
    wid = lax.axis_index("s") * NC + lax.axis_index("c")
    base = wid * FLAT

    pltpu.sync_copy(sf_hbm.at[pl.ds(base, FLAT)], idx_v)

    def fix(i, _):
        off = i * L
        pos = off + lax.iota(jnp.int32, L)
        f = lax.rem(pos, F)
        idx_v[pl.ds(off, L)] = idx_v[pl.ds(off, L)] + f * VOCAB_PER_FIELD
        return 0

    def fire(j, slot):
        pltpu.async_copy(
            table_hbm.at[idx_v.at[pl.ds(j * ROWS, ROWS)]],
            rows_v.at[slot],
            sems[slot],
        )

    def drain(j, slot):
        pltpu.make_async_copy(
            table_hbm.at[idx_v.at[pl.ds(j * ROWS, ROWS)]],
            rows_v.at[slot],
            sems[slot],
        ).wait()

    def accumulate(j, slot):
        G = 4

        def bag(b, _):
            r0 = b * F
            for d0 in range(0, D // L, G):
                ds_ = [pl.ds((d0 + g) * L, L) for g in range(G)]
                accs = [rows_v[slot, r0, dd] for dd in ds_]
                for f in range(1, F):
                    for g in range(G):
                        accs[g] = accs[g] + rows_v[slot, r0 + f, ds_[g]]
                for g in range(G):
                    out_v[j * C + b, ds_[g]] = accs[g] * (1.0 / F)
            return 0

        lax.fori_loop(0, C, bag, 0)

    VPC = -(-ROWS // L)
    lax.fori_loop(0, VPC, fix, 0)
    fire(0, 0)
    lax.fori_loop(VPC, FLAT // L, fix, 0)

    for s in range(1, NBUF - 1):
        fire(s, s)

    def ring(j4, _):
        j0 = NBUF * j4
        for s in range(NBUF):
            j = j0 + s

            @pl.when(j + NBUF - 1 < NCHUNK)
            def _():
                fire(j + NBUF - 1, (s + NBUF - 1) % NBUF)

            drain(j, s)
            accumulate(j, s)

            @pl.when(j == NCHUNK // 2 - 1)
            def _():
                pltpu.async_copy(
                    out_v.at[pl.ds(0, BPW // 2)],
                    out_hbm.at[pl.ds(wid * BPW, BPW // 2)],
                    sems[NBUF],
                )
        return 0

    lax.fori_loop(0, NCHUNK // NBUF, ring, 0)

    pltpu.sync_copy(
        out_v.at[pl.ds(BPW // 2, BPW // 2)],
        out_hbm.at[pl.ds(wid * BPW + BPW // 2, BPW // 2)],
    )
    pltpu.make_async_copy(
        out_v.at[pl.ds(0, BPW // 2)],
        out_hbm.at[pl.ds(wid * BPW, BPW // 2)],
        sems[NBUF],
    ).wait()


def kernel(sparse_features, table):
    sf_flat = sparse_features.astype(jnp.int32).reshape(-1)
    return _emb_bag(sf_flat, table)
